# baseline (speedup 1.0000x reference)
import functools

import jax
import jax.numpy as jnp
from jax import lax
from jax.experimental import pallas as pl
from jax.experimental.pallas import tpu as pltpu
from jax.experimental.pallas import tpu_sc as plsc

_NC = 2
_NS = 16
_NW = _NC * _NS
_LANES = 16


def _sc_mean_pool(x2, table):
    nchunks, ch = x2.shape
    _, d = table.shape
    nrows = nchunks // 2
    rows_per_w = nrows // _NW
    chunks_per_w = 2 * rows_per_w
    nvec = d // _LANES
    inv = 1.0 / (2.0 * ch)

    mesh = plsc.VectorSubcoreMesh(core_axis_name="c", subcore_axis_name="s")

    @functools.partial(
        pl.kernel,
        mesh=mesh,
        compiler_params=pltpu.CompilerParams(use_tc_tiling_on_sc=False),
        out_type=jax.ShapeDtypeStruct((nrows, d), jnp.float32),
        scratch_types=[
            pltpu.VMEM((chunks_per_w, ch), jnp.int32),
            pltpu.VMEM((ch, d), jnp.float32),
            pltpu.VMEM((ch, d), jnp.float32),
            pltpu.VMEM((ch, d), jnp.float32),
            pltpu.VMEM((ch, d), jnp.float32),
            pltpu.VMEM((rows_per_w, d), jnp.float32),
            pltpu.SemaphoreType.DMA,
            pltpu.SemaphoreType.DMA,
        ],
    )
    def pool(x_hbm, table_hbm, out_hbm, idx_v, b00, b01, b10, b11,
             pooled_v, sem0, sem1):
        cid = lax.axis_index("c")
        sid = lax.axis_index("s")
        wid = sid * _NC + cid
        base = wid * chunks_per_w
        pltpu.sync_copy(x_hbm.at[pl.ds(base, chunks_per_w)], idx_v)

        pltpu.async_copy(table_hbm.at[idx_v.at[0]], b00, sem0)
        pltpu.async_copy(table_hbm.at[idx_v.at[1]], b01, sem0)
        pltpu.async_copy(table_hbm.at[idx_v.at[2]], b10, sem1)
        pltpu.async_copy(table_hbm.at[idx_v.at[3]], b11, sem1)

        def do_row(row, ba, bb, sem):
            pltpu.make_async_copy(table_hbm.at[idx_v.at[0]], ba, sem).wait()
            pltpu.make_async_copy(table_hbm.at[idx_v.at[1]], bb, sem).wait()

            def red(buf):
                def body(r, acc):
                    return tuple(
                        acc[k] + buf[r, pl.ds(_LANES * k, _LANES)]
                        for k in range(nvec)
                    )
                return body

            zero = jnp.zeros((_LANES,), jnp.float32)
            acc = lax.fori_loop(0, ch, red(ba), (zero,) * nvec)
            acc = lax.fori_loop(0, ch, red(bb), acc)

            @pl.when(row + 2 < rows_per_w)
            def _():
                nxt = 2 * (row + 2)
                pltpu.async_copy(table_hbm.at[idx_v.at[nxt]], ba, sem)
                pltpu.async_copy(table_hbm.at[idx_v.at[nxt + 1]], bb, sem)

            for k in range(nvec):
                pooled_v[row, pl.ds(_LANES * k, _LANES)] = acc[k] * inv

        def outer(j, carry):
            do_row(2 * j, b00, b01, sem0)
            do_row(2 * j + 1, b10, b11, sem1)
            return carry

        lax.fori_loop(0, rows_per_w // 2, outer, 0)
        pltpu.sync_copy(pooled_v, out_hbm.at[pl.ds(wid * rows_per_w, rows_per_w)])

    return pool(x2, table)


def _tc_head(pooled, w, b):
    bn, d = pooled.shape
    n = w.shape[1]

    def body(p_ref, w_ref, b_ref, o_ref):
        z = jnp.dot(p_ref[...], w_ref[...],
                    preferred_element_type=jnp.float32) + b_ref[...]
        m = jnp.max(z, axis=1, keepdims=True)
        e = jnp.exp(z - m)
        o_ref[...] = e / jnp.sum(e, axis=1, keepdims=True)

    return pl.pallas_call(
        body,
        grid=(1,),
        in_specs=[
            pl.BlockSpec((bn, d), lambda i: (0, 0)),
            pl.BlockSpec((d, n), lambda i: (0, 0)),
            pl.BlockSpec((1, n), lambda i: (0, 0)),
        ],
        out_specs=pl.BlockSpec((bn, n), lambda i: (0, 0)),
        out_shape=jax.ShapeDtypeStruct((bn, n), jnp.float32),
    )(pooled, w, b.reshape(1, n))


def kernel(x, table, W, b):
    bsz, hist = x.shape
    x2 = x.reshape(2 * bsz, hist // 2)
    pooled = _sc_mean_pool(x2, table)
    return _tc_head(pooled, W, b)


# speedup vs baseline: 1.0539x; 1.0539x over previous
import functools

import jax
import jax.numpy as jnp
from jax import lax
from jax.experimental import pallas as pl
from jax.experimental.pallas import tpu as pltpu
from jax.experimental.pallas import tpu_sc as plsc

_NC = 2
_NS = 16
_NW = _NC * _NS
_LANES = 16


def _sc_mean_pool(x2, table):
    nchunks, ch = x2.shape
    _, d = table.shape
    nrows = nchunks // 2
    rows_per_w = nrows // _NW
    chunks_per_w = 2 * rows_per_w
    nvec = d // _LANES
    inv = 1.0 / (2.0 * ch)

    mesh = plsc.VectorSubcoreMesh(core_axis_name="c", subcore_axis_name="s")
    nbuf = 4
    unroll = 4
    assert ch % unroll == 0

    @functools.partial(
        pl.kernel,
        mesh=mesh,
        compiler_params=pltpu.CompilerParams(use_tc_tiling_on_sc=False),
        out_type=jax.ShapeDtypeStruct((nrows, d), jnp.float32),
        scratch_types=[
            pltpu.VMEM((chunks_per_w, ch), jnp.int32),
            [pltpu.VMEM((ch, d), jnp.float32)] * nbuf,
            pltpu.VMEM((rows_per_w, d), jnp.float32),
            [pltpu.SemaphoreType.DMA] * nbuf,
        ],
    )
    def pool(x_hbm, table_hbm, out_hbm, idx_v, bufs, pooled_v, sems):
        cid = lax.axis_index("c")
        sid = lax.axis_index("s")
        wid = sid * _NC + cid
        base = wid * chunks_per_w
        pltpu.sync_copy(x_hbm.at[pl.ds(base, chunks_per_w)], idx_v)

        for k in range(nbuf):
            pltpu.async_copy(table_hbm.at[idx_v.at[k]], bufs[k], sems[k])

        def chunk_sum(buf, acc):
            def body(i, acc):
                r = unroll * i
                out = []
                for k in range(nvec):
                    sl = pl.ds(_LANES * k, _LANES)
                    v0 = buf[r, sl] + buf[r + 1, sl]
                    v1 = buf[r + 2, sl] + buf[r + 3, sl]
                    out.append(acc[k] + (v0 + v1))
                return tuple(out)
            return lax.fori_loop(0, ch // unroll, body, acc)

        zero = jnp.zeros((_LANES,), jnp.float32)

        def outer(j, carry):
            for h in range(2):
                row = 2 * j + h
                acc = (zero,) * nvec
                for q in range(2):
                    k = 2 * h + q
                    t = nbuf * j + k
                    pltpu.make_async_copy(
                        table_hbm.at[idx_v.at[0]], bufs[k], sems[k]).wait()
                    acc = chunk_sum(bufs[k], acc)

                    @pl.when(t + nbuf < chunks_per_w)
                    def _():
                        pltpu.async_copy(
                            table_hbm.at[idx_v.at[t + nbuf]], bufs[k], sems[k])

                for k in range(nvec):
                    pooled_v[row, pl.ds(_LANES * k, _LANES)] = acc[k] * inv
            return carry

        lax.fori_loop(0, rows_per_w // 2, outer, 0)
        pltpu.sync_copy(pooled_v, out_hbm.at[pl.ds(wid * rows_per_w, rows_per_w)])

    return pool(x2, table)


def _tc_head(pooled, w, b):
    bn, d = pooled.shape
    n = w.shape[1]

    def body(p_ref, w_ref, b_ref, o_ref):
        z = jnp.dot(p_ref[...], w_ref[...],
                    preferred_element_type=jnp.float32) + b_ref[...]
        m = jnp.max(z, axis=1, keepdims=True)
        e = jnp.exp(z - m)
        o_ref[...] = e / jnp.sum(e, axis=1, keepdims=True)

    return pl.pallas_call(
        body,
        grid=(1,),
        in_specs=[
            pl.BlockSpec((bn, d), lambda i: (0, 0)),
            pl.BlockSpec((d, n), lambda i: (0, 0)),
            pl.BlockSpec((1, n), lambda i: (0, 0)),
        ],
        out_specs=pl.BlockSpec((bn, n), lambda i: (0, 0)),
        out_shape=jax.ShapeDtypeStruct((bn, n), jnp.float32),
    )(pooled, w, b.reshape(1, n))


def kernel(x, table, W, b):
    bsz, hist = x.shape
    x2 = x.reshape(2 * bsz, hist // 2)
    pooled = _sc_mean_pool(x2, table)
    return _tc_head(pooled, W, b)


# speedup vs baseline: 1.0780x; 1.0229x over previous
import functools

import jax
import jax.numpy as jnp
from jax import lax
from jax.experimental import pallas as pl
from jax.experimental.pallas import tpu as pltpu
from jax.experimental.pallas import tpu_sc as plsc

_NC = 2
_NS = 16
_NW = _NC * _NS
_LANES = 16


def _sc_mean_pool(x2, table):
    nchunks, ch = x2.shape
    _, d = table.shape
    nrows = nchunks // 2
    rows_per_w = nrows // _NW
    chunks_per_w = 2 * rows_per_w
    nvec = d // _LANES
    inv = 1.0 / (2.0 * ch)

    mesh = plsc.VectorSubcoreMesh(core_axis_name="c", subcore_axis_name="s")
    nbuf = 8
    unroll = 4
    assert ch % unroll == 0 and chunks_per_w % nbuf == 0

    @functools.partial(
        pl.kernel,
        mesh=mesh,
        compiler_params=pltpu.CompilerParams(use_tc_tiling_on_sc=False),
        out_type=jax.ShapeDtypeStruct((nrows, d), jnp.float32),
        scratch_types=[
            pltpu.VMEM((chunks_per_w, ch), jnp.int32),
            [pltpu.VMEM((ch, d), jnp.float32)] * nbuf,
            pltpu.VMEM((rows_per_w, d), jnp.float32),
            [pltpu.SemaphoreType.DMA] * nbuf,
        ],
    )
    def pool(x_hbm, table_hbm, out_hbm, idx_v, bufs, pooled_v, sems):
        cid = lax.axis_index("c")
        sid = lax.axis_index("s")
        wid = sid * _NC + cid
        base = wid * chunks_per_w
        pltpu.sync_copy(x_hbm.at[pl.ds(base, chunks_per_w)], idx_v)

        for k in range(nbuf):
            pltpu.async_copy(table_hbm.at[idx_v.at[k]], bufs[k], sems[k])

        def chunk_sum(buf, acc):
            def body(i, acc):
                r = unroll * i
                out = []
                for k in range(nvec):
                    sl = pl.ds(_LANES * k, _LANES)
                    v0 = buf[r, sl] + buf[r + 1, sl]
                    v1 = buf[r + 2, sl] + buf[r + 3, sl]
                    out.append(acc[k] + (v0 + v1))
                return tuple(out)
            return lax.fori_loop(0, ch // unroll, body, acc)

        zero = jnp.zeros((_LANES,), jnp.float32)

        def outer(j, carry):
            acc = (zero,) * nvec
            for u in range(nbuf):
                t = nbuf * j + u
                pltpu.make_async_copy(
                    table_hbm.at[idx_v.at[0]], bufs[u], sems[u]).wait()
                acc = chunk_sum(bufs[u], acc)

                @pl.when(t + nbuf < chunks_per_w)
                def _():
                    pltpu.async_copy(
                        table_hbm.at[idx_v.at[t + nbuf]], bufs[u], sems[u])

                if u % 2 == 1:
                    row = (nbuf * j + u) // 2
                    for k in range(nvec):
                        pooled_v[row, pl.ds(_LANES * k, _LANES)] = acc[k] * inv
                    acc = (zero,) * nvec
            return carry

        lax.fori_loop(0, chunks_per_w // nbuf, outer, 0)
        pltpu.sync_copy(pooled_v, out_hbm.at[pl.ds(wid * rows_per_w, rows_per_w)])

    return pool(x2, table)


def _tc_head(pooled, w, b):
    bn, d = pooled.shape
    n = w.shape[1]

    def body(p_ref, w_ref, b_ref, o_ref):
        z = jnp.dot(p_ref[...], w_ref[...],
                    preferred_element_type=jnp.float32) + b_ref[...]
        m = jnp.max(z, axis=1, keepdims=True)
        e = jnp.exp(z - m)
        o_ref[...] = e / jnp.sum(e, axis=1, keepdims=True)

    return pl.pallas_call(
        body,
        grid=(1,),
        in_specs=[
            pl.BlockSpec((bn, d), lambda i: (0, 0)),
            pl.BlockSpec((d, n), lambda i: (0, 0)),
            pl.BlockSpec((1, n), lambda i: (0, 0)),
        ],
        out_specs=pl.BlockSpec((bn, n), lambda i: (0, 0)),
        out_shape=jax.ShapeDtypeStruct((bn, n), jnp.float32),
    )(pooled, w, b.reshape(1, n))


def kernel(x, table, W, b):
    bsz, hist = x.shape
    x2 = x.reshape(2 * bsz, hist // 2)
    pooled = _sc_mean_pool(x2, table)
    return _tc_head(pooled, W, b)


# speedup vs baseline: 1.0818x; 1.0035x over previous
import functools

import jax
import jax.numpy as jnp
from jax import lax
from jax.experimental import pallas as pl
from jax.experimental.pallas import tpu as pltpu
from jax.experimental.pallas import tpu_sc as plsc

_NC = 2
_NS = 16
_NW = _NC * _NS
_LANES = 16


def _sc_mean_pool(xf, table, hist):
    (ntot,) = xf.shape
    _, d = table.shape
    nrows = ntot // hist
    rows_per_w = nrows // _NW
    idx_per_w = rows_per_w * hist
    nvec = d // _LANES
    inv = 1.0 / float(hist)

    mesh = plsc.VectorSubcoreMesh(core_axis_name="c", subcore_axis_name="s")
    nbuf = 4
    unroll = 4
    assert hist % unroll == 0 and rows_per_w % nbuf == 0

    @functools.partial(
        pl.kernel,
        mesh=mesh,
        compiler_params=pltpu.CompilerParams(use_tc_tiling_on_sc=False),
        out_type=jax.ShapeDtypeStruct((nrows, d), jnp.float32),
        scratch_types=[
            pltpu.VMEM((idx_per_w,), jnp.int32),
            [pltpu.VMEM((hist, d), jnp.float32)] * nbuf,
            pltpu.VMEM((rows_per_w, d), jnp.float32),
            [pltpu.SemaphoreType.DMA] * nbuf,
        ],
    )
    def pool(x_hbm, table_hbm, out_hbm, idx_v, bufs, pooled_v, sems):
        cid = lax.axis_index("c")
        sid = lax.axis_index("s")
        wid = sid * _NC + cid
        pltpu.sync_copy(x_hbm.at[pl.ds(wid * idx_per_w, idx_per_w)], idx_v)

        def gather_row(r, buf, sem):
            pltpu.async_copy(
                table_hbm.at[idx_v.at[pl.ds(r * hist, hist)]], buf, sem)

        for k in range(nbuf):
            gather_row(k, bufs[k], sems[k])

        def chunk_sum(buf, acc):
            def body(i, acc):
                r = unroll * i
                out = []
                for k in range(nvec):
                    sl = pl.ds(_LANES * k, _LANES)
                    v0 = buf[r, sl] + buf[r + 1, sl]
                    v1 = buf[r + 2, sl] + buf[r + 3, sl]
                    out.append(acc[k] + (v0 + v1))
                return tuple(out)
            return lax.fori_loop(0, hist // unroll, body, acc)

        zero = jnp.zeros((_LANES,), jnp.float32)

        def outer(j, carry):
            for u in range(nbuf):
                r = nbuf * j + u
                pltpu.make_async_copy(
                    table_hbm.at[idx_v.at[pl.ds(0, hist)]],
                    bufs[u], sems[u]).wait()
                acc = chunk_sum(bufs[u], (zero,) * nvec)

                @pl.when(r + nbuf < rows_per_w)
                def _():
                    gather_row(r + nbuf, bufs[u], sems[u])

                for k in range(nvec):
                    pooled_v[r, pl.ds(_LANES * k, _LANES)] = acc[k] * inv
            return carry

        lax.fori_loop(0, rows_per_w // nbuf, outer, 0)
        pltpu.sync_copy(pooled_v, out_hbm.at[pl.ds(wid * rows_per_w, rows_per_w)])

    return pool(xf, table)


def _tc_head(pooled, w, b):
    bn, d = pooled.shape
    n = w.shape[1]

    def body(p_ref, w_ref, b_ref, o_ref):
        z = jnp.dot(p_ref[...], w_ref[...],
                    preferred_element_type=jnp.float32) + b_ref[...]
        m = jnp.max(z, axis=1, keepdims=True)
        e = jnp.exp(z - m)
        o_ref[...] = e / jnp.sum(e, axis=1, keepdims=True)

    return pl.pallas_call(
        body,
        grid=(1,),
        in_specs=[
            pl.BlockSpec((bn, d), lambda i: (0, 0)),
            pl.BlockSpec((d, n), lambda i: (0, 0)),
            pl.BlockSpec((1, n), lambda i: (0, 0)),
        ],
        out_specs=pl.BlockSpec((bn, n), lambda i: (0, 0)),
        out_shape=jax.ShapeDtypeStruct((bn, n), jnp.float32),
    )(pooled, w, b.reshape(1, n))


def kernel(x, table, W, b):
    bsz, hist = x.shape
    pooled = _sc_mean_pool(x.reshape(-1), table, hist)
    return _tc_head(pooled, W, b)
